# baseline (speedup 1.0000x reference)
import dataclasses
import functools

import jax
import jax.numpy as jnp
from jax import lax
from jax.experimental import pallas as pl
from jax.experimental.pallas import tpu as pltpu
from jax.experimental.pallas import tpu_sc as plsc

N = 10000
E = 320000
IN_CH = 128
HID = 256
HALF = 128

NC = 2
NS = 16
ROWS_PER_SUB = N // NS
EDGES_PER_SUB = E // NS
EDGES_PER_WORKER = E // (NC * NS)
BLK = 128
NFULL = EDGES_PER_SUB // BLK
REM = EDGES_PER_SUB - NFULL * BLK
AGG_ROWS = 632
AGG_ROWS_LAST = N - 15 * AGG_ROWS
DEG_BLK = 128
DEG_NBLK = EDGES_PER_WORKER // DEG_BLK
DEG_REM = EDGES_PER_WORKER - DEG_NBLK * DEG_BLK

ROW_BLK = 1000
GRID = N // ROW_BLK

_f32 = jnp.float32


@functools.lru_cache(maxsize=None)
def _sc_mesh():
    return plsc.VectorSubcoreMesh(
        core_axis_name="c", subcore_axis_name="s", num_cores=NC, num_subcores=NS
    )


DEG_ROWS = 632
DEG_ROWS_LAST = N - 15 * DEG_ROWS


def _deg_body(dst_hbm, zeros_hbm, deg0_hbm, deg1_hbm, acc, ones_v,
              idx_a, idx_b, idx_r, buf_v, sem_a, sem_b):
    c = lax.axis_index("c")
    s = lax.axis_index("s")
    wid = c * NS + s
    idxb = (idx_a, idx_b)
    sem_i = (sem_a, sem_b)
    for k in range(DEG_BLK // 16):
        ones_v[pl.ds(16 * k, 16)] = jnp.full((16,), 1.0, _f32)

    def rows_foreach(fn):
        @pl.when(s < 15)
        def _():
            fn(pl.ds(s * DEG_ROWS, DEG_ROWS), DEG_ROWS)

        @pl.when(s == 15)
        def _():
            fn(pl.ds(15 * DEG_ROWS, DEG_ROWS_LAST), DEG_ROWS_LAST)

    def zero_rows(rsl, nrows):
        pltpu.sync_copy(zeros_hbm.at[pl.ds(0, nrows)], buf_v.at[pl.ds(0, nrows)])
        pltpu.sync_copy(buf_v.at[pl.ds(0, nrows)], acc.at[rsl])

    rows_foreach(zero_rows)
    plsc.subcore_barrier()
    e0 = wid * EDGES_PER_WORKER

    def issue_idx(j, b):
        pltpu.async_copy(
            dst_hbm.at[pl.ds(e0 + j * DEG_BLK, DEG_BLK)], idxb[b], sem_i[b])

    def wait_idx(j, b):
        pltpu.make_async_copy(
            dst_hbm.at[pl.ds(e0 + j * DEG_BLK, DEG_BLK)], idxb[b],
            sem_i[b]).wait()

    def scatter_ones(b):
        pltpu.sync_copy(ones_v, acc.at[idxb[b]], add=True)

    issue_idx(0, 0)
    issue_idx(1, 1)

    @pl.loop(0, DEG_NBLK // 2 - 1)
    def _(i):
        j = 2 * i
        wait_idx(j, 0)
        scatter_ones(0)
        issue_idx(j + 2, 0)
        wait_idx(j + 1, 1)
        scatter_ones(1)
        issue_idx(j + 3, 1)

    wait_idx(DEG_NBLK - 2, 0)
    scatter_ones(0)
    wait_idx(DEG_NBLK - 1, 1)
    scatter_ones(1)

    pltpu.sync_copy(
        dst_hbm.at[pl.ds(e0 + DEG_NBLK * DEG_BLK, DEG_REM)], idx_r)
    pltpu.sync_copy(ones_v.at[pl.ds(0, DEG_REM)], acc.at[idx_r], add=True)

    plsc.subcore_barrier()

    def out_rows(dst_ref):
        def fn(rsl, nrows):
            pltpu.sync_copy(acc.at[rsl], buf_v.at[pl.ds(0, nrows)])
            pltpu.sync_copy(buf_v.at[pl.ds(0, nrows)], dst_ref.at[rsl])

        rows_foreach(fn)

    @pl.when(c == 0)
    def _():
        out_rows(deg0_hbm)

    @pl.when(c == 1)
    def _():
        out_rows(deg1_hbm)


@functools.lru_cache(maxsize=None)
def _deg_call_built():
    return pl.kernel(
        _deg_body,
        out_type=[jax.ShapeDtypeStruct((N,), _f32),
                  jax.ShapeDtypeStruct((N,), _f32)],
        mesh=_sc_mesh(),
        scratch_types=[
            pltpu.VMEM_SHARED((N,), _f32),
            pltpu.VMEM((DEG_BLK,), _f32),
            pltpu.VMEM((DEG_BLK,), jnp.int32),
            pltpu.VMEM((DEG_BLK,), jnp.int32),
            pltpu.VMEM((DEG_REM,), jnp.int32),
            pltpu.VMEM((DEG_ROWS,), _f32),
            pltpu.SemaphoreType.DMA,
            pltpu.SemaphoreType.DMA,
        ],
    )


def _deg_call(*args):
    return _deg_call_built()(*args)


def _agg_body(u0, u1, src_hbm, dst_hbm, zeros_hbm, out0, out1,
              acc, src_a, src_b, dst_a, dst_b, dstp_a, dstp_b,
              rows16_a, rows16_b, rowsf_a, rowsf_b, src_r, dst_r,
              sem_ia, sem_ib, sem_ga, sem_gb, sem_sa, sem_sb):
    c = lax.axis_index("c")
    s = lax.axis_index("s")
    srcb = (src_a, src_b)
    dstb = (dst_a, dst_b)
    dstp = (dstp_a, dstp_b)
    rows16 = (rows16_a, rows16_b)
    rowsf = (rowsf_a, rowsf_b)
    sem_i = (sem_ia, sem_ib)
    sem_g = (sem_ga, sem_gb)
    sem_s = (sem_sa, sem_sb)
    rows_v = rowsf_a

    def rows_foreach(fn):
        @pl.when(s < 15)
        def _():
            fn(s * AGG_ROWS, AGG_ROWS)

        @pl.when(s == 15)
        def _():
            fn(15 * AGG_ROWS, AGG_ROWS_LAST)

    def chunks(nrows):
        off = 0
        while off < nrows:
            k = min(BLK, nrows - off)
            yield off, k
            off += k

    def zero_rows(base, nrows):
        pltpu.sync_copy(zeros_hbm.at[pl.ds(0, BLK)], rows_v)
        for off, k in chunks(nrows):
            pltpu.sync_copy(rows_v.at[pl.ds(0, k)], acc.at[pl.ds(base + off, k)])

    rows_foreach(zero_rows)
    plsc.subcore_barrier()
    e0 = s * EDGES_PER_SUB

    def run(u_hbm):
        def issue_idx(j, b):
            off = e0 + j * BLK
            pltpu.async_copy(src_hbm.at[pl.ds(off, BLK)], srcb[b], sem_i[b])
            pltpu.async_copy(dst_hbm.at[pl.ds(off, BLK)], dstb[b], sem_i[b])

        def wait_idx(j, b):
            off = e0 + j * BLK
            pltpu.make_async_copy(
                src_hbm.at[pl.ds(off, BLK)], srcb[b], sem_i[b]).wait()
            pltpu.make_async_copy(
                dst_hbm.at[pl.ds(off, BLK)], dstb[b], sem_i[b]).wait()

        def copy_dst(b):
            for k in range(BLK // 16):
                sl = pl.ds(16 * k, 16)
                dstp[b][sl] = dstb[b][sl]

        def issue_gather(b):
            pltpu.async_copy(u_hbm.at[srcb[b]], rows16[b], sem_g[b])

        def wait_gather(b):
            pltpu.make_async_copy(u_hbm.at[srcb[b]], rows16[b], sem_g[b]).wait()

        def expand(b, nrows):
            @pl.loop(0, nrows, unroll=4)
            def _(r):
                for k in range(HALF // 32):
                    w = rows16[b][r, pl.ds(16 * k, 16)]
                    rowsf[b][r, pl.ds(16 * k, 16)] = plsc.bitcast(
                        jnp.left_shift(w, jnp.int32(16)), _f32)
                    rowsf[b][r, pl.ds(64 + 16 * k, 16)] = plsc.bitcast(
                        jnp.bitwise_and(w, jnp.int32(-65536)), _f32)

        def issue_scatter(b):
            pltpu.async_copy(rowsf[b], acc.at[dstp[b]], sem_s[b], add=True)

        def wait_scatter(b):
            pltpu.make_async_copy(rowsf[b], acc.at[dstp[b]], sem_s[b]).wait()

        def step(j, b, *, first=False, last=False):
            if not first:
                wait_scatter(b)
            wait_idx(j, b)
            copy_dst(b)
            issue_gather(b)
            wait_gather(1 - b)
            expand(1 - b, BLK)
            issue_scatter(1 - b)
            if not last:
                issue_idx(j + 1, 1 - b)

        issue_idx(0, 0)
        wait_idx(0, 0)
        copy_dst(0)
        issue_gather(0)
        issue_idx(1, 1)
        step(1, 1, first=True)

        @pl.loop(0, (NFULL - 4) // 2)
        def _(i):
            j = 2 * i + 2
            step(j, 0)
            step(j + 1, 1)

        step(NFULL - 2, 0)
        step(NFULL - 1, 1, last=True)
        wait_gather(1)
        expand(1, BLK)
        issue_scatter(1)
        wait_scatter(0)
        wait_scatter(1)

        off = e0 + NFULL * BLK
        pltpu.sync_copy(src_hbm.at[pl.ds(off, REM)], src_r)
        pltpu.sync_copy(dst_hbm.at[pl.ds(off, REM)], dst_r)
        pltpu.async_copy(
            u_hbm.at[src_r], rows16_a.at[pl.ds(0, REM)], sem_ga).wait()
        expand(0, REM)
        pltpu.sync_copy(rowsf_a.at[pl.ds(0, REM)], acc.at[dst_r], add=True)

    @pl.when(c == 0)
    def _():
        run(u0)

    @pl.when(c == 1)
    def _():
        run(u1)

    plsc.subcore_barrier()

    def drain(out_ref):
        def fn(base, nrows):
            for off, k in chunks(nrows):
                rsl = pl.ds(base + off, k)
                pltpu.sync_copy(acc.at[rsl], rows_v.at[pl.ds(0, k)])
                pltpu.sync_copy(rows_v.at[pl.ds(0, k)], out_ref.at[rsl])

        rows_foreach(fn)

    @pl.when(c == 0)
    def _():
        drain(out0)

    @pl.when(c == 1)
    def _():
        drain(out1)


def _sc_compiler_params():
    cp = pltpu.CompilerParams()
    if "needs_layout_passes" in pltpu.CompilerParams.__dataclass_fields__:
        cp = dataclasses.replace(cp, needs_layout_passes=False)
    if "use_tc_tiling_on_sc" in pltpu.CompilerParams.__dataclass_fields__:
        cp = dataclasses.replace(cp, use_tc_tiling_on_sc=False)
    return cp


@functools.lru_cache(maxsize=None)
def _agg_call_built():
    return pl.kernel(
        _agg_body,
        compiler_params=_sc_compiler_params(),
        out_type=[jax.ShapeDtypeStruct((N, HALF), _f32),
                  jax.ShapeDtypeStruct((N, HALF), _f32)],
        mesh=_sc_mesh(),
        scratch_types=[
            pltpu.VMEM_SHARED((N, HALF), _f32),
            pltpu.VMEM((BLK,), jnp.int32),
            pltpu.VMEM((BLK,), jnp.int32),
            pltpu.VMEM((BLK,), jnp.int32),
            pltpu.VMEM((BLK,), jnp.int32),
            pltpu.VMEM((BLK,), jnp.int32),
            pltpu.VMEM((BLK,), jnp.int32),
            pltpu.VMEM((BLK, HALF // 2), jnp.int32),
            pltpu.VMEM((BLK, HALF // 2), jnp.int32),
            pltpu.VMEM((BLK, HALF), _f32),
            pltpu.VMEM((BLK, HALF), _f32),
            pltpu.VMEM((REM,), jnp.int32),
            pltpu.VMEM((REM,), jnp.int32),
            pltpu.SemaphoreType.DMA,
            pltpu.SemaphoreType.DMA,
            pltpu.SemaphoreType.DMA,
            pltpu.SemaphoreType.DMA,
            pltpu.SemaphoreType.DMA,
            pltpu.SemaphoreType.DMA,
        ],
    )


def _agg_call(*args):
    return _agg_call_built()(*args)


_DOT = functools.partial(
    lax.dot_general,
    dimension_numbers=(((1,), (0,)), ((), ())),
    precision=lax.Precision.HIGHEST,
    preferred_element_type=_f32,
)


def _mm1_body(x_ref, w_ref, o_ref):
    o_ref[...] = _DOT(x_ref[...], w_ref[...])


def _mm1(x, w):
    return pl.pallas_call(
        _mm1_body,
        grid=(GRID,),
        in_specs=[pl.BlockSpec((ROW_BLK, IN_CH), lambda i: (i, 0)),
                  pl.BlockSpec((IN_CH, HID), lambda i: (0, 0))],
        out_specs=pl.BlockSpec((ROW_BLK, HID), lambda i: (i, 0)),
        out_shape=jax.ShapeDtypeStruct((N, HID), _f32),
    )(x, w)


def _pack_half(uh):
    b = lax.bitcast_convert_type(
        uh.astype(jnp.bfloat16).astype(_f32), jnp.int32)
    lo = jnp.bitwise_and(jnp.right_shift(b[:, :HALF // 2], 16),
                         jnp.int32(0xFFFF))
    hi = jnp.bitwise_and(b[:, HALF // 2:], jnp.int32(-65536))
    return jnp.bitwise_or(lo, hi)


def _scale_body(d0_ref, d1_ref, v_ref, dis_ref, u0_ref, u1_ref):
    deg = d0_ref[...] + d1_ref[...]
    dis = jnp.where(deg > 0, lax.rsqrt(jnp.maximum(deg, 1e-12)), 0.0)
    dis_ref[...] = dis
    u = dis * v_ref[...]
    u0_ref[...] = _pack_half(u[:, :HALF])
    u1_ref[...] = _pack_half(u[:, HALF:])


def _scale(deg0, deg1, v):
    return pl.pallas_call(
        _scale_body,
        grid=(GRID,),
        in_specs=[pl.BlockSpec((ROW_BLK, 1), lambda i: (i, 0)),
                  pl.BlockSpec((ROW_BLK, 1), lambda i: (i, 0)),
                  pl.BlockSpec((ROW_BLK, HID), lambda i: (i, 0))],
        out_specs=[pl.BlockSpec((ROW_BLK, 1), lambda i: (i, 0)),
                   pl.BlockSpec((ROW_BLK, HALF // 2), lambda i: (i, 0)),
                   pl.BlockSpec((ROW_BLK, HALF // 2), lambda i: (i, 0))],
        out_shape=[jax.ShapeDtypeStruct((N, 1), _f32),
                   jax.ShapeDtypeStruct((N, HALF // 2), jnp.int32),
                   jax.ShapeDtypeStruct((N, HALF // 2), jnp.int32)],
    )(deg0, deg1, v)


def _elu(x):
    return jnp.where(x > 0, x, jnp.exp(jnp.minimum(x, 0.0)) - 1.0)


def _mid_body(a0_ref, a1_ref, dis_ref, b_ref, w_ref, u0_ref, u1_ref):
    dis = dis_ref[...]
    h = jnp.concatenate([a0_ref[...], a1_ref[...]], axis=1)
    h = _elu(dis * h + b_ref[...])
    u = dis * _DOT(h, w_ref[...])
    u0_ref[...] = _pack_half(u[:, :HALF])
    u1_ref[...] = _pack_half(u[:, HALF:])


def _mid(a0, a1, dis, b, w):
    return pl.pallas_call(
        _mid_body,
        grid=(GRID,),
        in_specs=[pl.BlockSpec((ROW_BLK, HALF), lambda i: (i, 0)),
                  pl.BlockSpec((ROW_BLK, HALF), lambda i: (i, 0)),
                  pl.BlockSpec((ROW_BLK, 1), lambda i: (i, 0)),
                  pl.BlockSpec((1, HID), lambda i: (0, 0)),
                  pl.BlockSpec((HID, HID), lambda i: (0, 0))],
        out_specs=[pl.BlockSpec((ROW_BLK, HALF // 2), lambda i: (i, 0)),
                   pl.BlockSpec((ROW_BLK, HALF // 2), lambda i: (i, 0))],
        out_shape=[jax.ShapeDtypeStruct((N, HALF // 2), jnp.int32),
                   jax.ShapeDtypeStruct((N, HALF // 2), jnp.int32)],
    )(a0, a1, dis, b, w)


def _final_body(a0_ref, a1_ref, dis_ref, b_ref, o_ref):
    dis = dis_ref[...]
    h = jnp.concatenate([a0_ref[...], a1_ref[...]], axis=1)
    o_ref[...] = _elu(dis * h + b_ref[...])


def _final(a0, a1, dis, b):
    return pl.pallas_call(
        _final_body,
        grid=(GRID,),
        in_specs=[pl.BlockSpec((ROW_BLK, HALF), lambda i: (i, 0)),
                  pl.BlockSpec((ROW_BLK, HALF), lambda i: (i, 0)),
                  pl.BlockSpec((ROW_BLK, 1), lambda i: (i, 0)),
                  pl.BlockSpec((1, HID), lambda i: (0, 0))],
        out_specs=pl.BlockSpec((ROW_BLK, HID), lambda i: (i, 0)),
        out_shape=jax.ShapeDtypeStruct((N, HID), _f32),
    )(a0, a1, dis, b)


def kernel(x, edge_index, W1, b1, W2, b2, W3, b3):
    ei = edge_index.astype(jnp.int32)
    src = ei[0]
    dst = ei[1]
    zeros1 = jnp.zeros((N,), _f32)
    zeros2 = jnp.zeros((N, HALF), _f32)

    deg0, deg1 = _deg_call(dst, zeros1)
    v1 = _mm1(x, W1)
    dis, u0, u1 = _scale(deg0.reshape(N, 1), deg1.reshape(N, 1), v1)

    a0, a1 = _agg_call(u0, u1, src, dst, zeros2)
    u0, u1 = _mid(a0, a1, dis, b1.reshape(1, HID), W2)
    a0, a1 = _agg_call(u0, u1, src, dst, zeros2)
    u0, u1 = _mid(a0, a1, dis, b2.reshape(1, HID), W3)
    a0, a1 = _agg_call(u0, u1, src, dst, zeros2)
    return _final(a0, a1, dis, b3.reshape(1, HID))


# speedup vs baseline: 2.2589x; 2.2589x over previous
import functools

import jax
import jax.numpy as jnp
from jax import lax
from jax.experimental import pallas as pl
from jax.experimental.pallas import tpu as pltpu
from jax.experimental.pallas import tpu_sc as plsc

N = 10000
E = 320000
IN_CH = 128
HID = 256
HALF = 128

NC = 2
NS = 16
ROWS_PER_SUB = N // NS
EDGES_PER_SUB = E // NS
EDGES_PER_WORKER = E // (NC * NS)
BLK = 128
NFULL = EDGES_PER_SUB // BLK
REM = EDGES_PER_SUB - NFULL * BLK
AGG_ROWS = 632
AGG_ROWS_LAST = N - 15 * AGG_ROWS
DEG_BLK = 128
DEG_NBLK = EDGES_PER_WORKER // DEG_BLK
DEG_REM = EDGES_PER_WORKER - DEG_NBLK * DEG_BLK

ROW_BLK = 1000
GRID = N // ROW_BLK

_f32 = jnp.float32


@functools.lru_cache(maxsize=None)
def _sc_mesh():
    return plsc.VectorSubcoreMesh(
        core_axis_name="c", subcore_axis_name="s", num_cores=NC, num_subcores=NS
    )


DEG_ROWS = 632
DEG_ROWS_LAST = N - 15 * DEG_ROWS


def _deg_body(dst_hbm, zeros_hbm, deg0_hbm, deg1_hbm, acc, ones_v,
              idx_a, idx_b, idx_r, buf_v, sem_a, sem_b):
    c = lax.axis_index("c")
    s = lax.axis_index("s")
    wid = c * NS + s
    idxb = (idx_a, idx_b)
    sem_i = (sem_a, sem_b)
    for k in range(DEG_BLK // 16):
        ones_v[pl.ds(16 * k, 16)] = jnp.full((16,), 1.0, _f32)

    def rows_foreach(fn):
        @pl.when(s < 15)
        def _():
            fn(pl.ds(s * DEG_ROWS, DEG_ROWS), DEG_ROWS)

        @pl.when(s == 15)
        def _():
            fn(pl.ds(15 * DEG_ROWS, DEG_ROWS_LAST), DEG_ROWS_LAST)

    def zero_rows(rsl, nrows):
        pltpu.sync_copy(zeros_hbm.at[pl.ds(0, nrows)], buf_v.at[pl.ds(0, nrows)])
        pltpu.sync_copy(buf_v.at[pl.ds(0, nrows)], acc.at[rsl])

    rows_foreach(zero_rows)
    plsc.subcore_barrier()
    e0 = wid * EDGES_PER_WORKER

    def issue_idx(j, b):
        pltpu.async_copy(
            dst_hbm.at[pl.ds(e0 + j * DEG_BLK, DEG_BLK)], idxb[b], sem_i[b])

    def wait_idx(j, b):
        pltpu.make_async_copy(
            dst_hbm.at[pl.ds(e0 + j * DEG_BLK, DEG_BLK)], idxb[b],
            sem_i[b]).wait()

    def scatter_ones(b):
        pltpu.sync_copy(ones_v, acc.at[idxb[b]], add=True)

    issue_idx(0, 0)
    issue_idx(1, 1)

    @pl.loop(0, DEG_NBLK // 2 - 1)
    def _(i):
        j = 2 * i
        wait_idx(j, 0)
        scatter_ones(0)
        issue_idx(j + 2, 0)
        wait_idx(j + 1, 1)
        scatter_ones(1)
        issue_idx(j + 3, 1)

    wait_idx(DEG_NBLK - 2, 0)
    scatter_ones(0)
    wait_idx(DEG_NBLK - 1, 1)
    scatter_ones(1)

    pltpu.sync_copy(
        dst_hbm.at[pl.ds(e0 + DEG_NBLK * DEG_BLK, DEG_REM)], idx_r)
    pltpu.sync_copy(ones_v.at[pl.ds(0, DEG_REM)], acc.at[idx_r], add=True)

    plsc.subcore_barrier()

    def out_rows(dst_ref):
        def fn(rsl, nrows):
            pltpu.sync_copy(acc.at[rsl], buf_v.at[pl.ds(0, nrows)])
            pltpu.sync_copy(buf_v.at[pl.ds(0, nrows)], dst_ref.at[rsl])

        rows_foreach(fn)

    @pl.when(c == 0)
    def _():
        out_rows(deg0_hbm)

    @pl.when(c == 1)
    def _():
        out_rows(deg1_hbm)


@functools.lru_cache(maxsize=None)
def _deg_call_built():
    return pl.kernel(
        _deg_body,
        out_type=[jax.ShapeDtypeStruct((N,), _f32),
                  jax.ShapeDtypeStruct((N,), _f32)],
        mesh=_sc_mesh(),
        scratch_types=[
            pltpu.VMEM_SHARED((N,), _f32),
            pltpu.VMEM((DEG_BLK,), _f32),
            pltpu.VMEM((DEG_BLK,), jnp.int32),
            pltpu.VMEM((DEG_BLK,), jnp.int32),
            pltpu.VMEM((DEG_REM,), jnp.int32),
            pltpu.VMEM((DEG_ROWS,), _f32),
            pltpu.SemaphoreType.DMA,
            pltpu.SemaphoreType.DMA,
        ],
    )


def _deg_call(*args):
    return _deg_call_built()(*args)


def _agg_body(u0, u1, src_hbm, dst_hbm, zeros_hbm, out0, out1,
              acc, src_a, src_b, dst_a, dst_b, dstp_a, dstp_b,
              rows_a, rows_b, src_r, dst_r,
              sem_ia, sem_ib, sem_ga, sem_gb, sem_sa, sem_sb):
    c = lax.axis_index("c")
    s = lax.axis_index("s")
    srcb = (src_a, src_b)
    dstb = (dst_a, dst_b)
    dstp = (dstp_a, dstp_b)
    rows = (rows_a, rows_b)
    sem_i = (sem_ia, sem_ib)
    sem_g = (sem_ga, sem_gb)
    sem_s = (sem_sa, sem_sb)
    rows_v = rows_a

    def rows_foreach(fn):
        @pl.when(s < 15)
        def _():
            fn(s * AGG_ROWS, AGG_ROWS)

        @pl.when(s == 15)
        def _():
            fn(15 * AGG_ROWS, AGG_ROWS_LAST)

    def chunks(nrows):
        off = 0
        while off < nrows:
            k = min(BLK, nrows - off)
            yield off, k
            off += k

    def zero_rows(base, nrows):
        pltpu.sync_copy(zeros_hbm.at[pl.ds(0, BLK)], rows_v)
        for off, k in chunks(nrows):
            pltpu.sync_copy(rows_v.at[pl.ds(0, k)], acc.at[pl.ds(base + off, k)])

    rows_foreach(zero_rows)
    plsc.subcore_barrier()
    e0 = s * EDGES_PER_SUB

    def run(u_hbm):
        def issue_idx(j, b):
            off = e0 + j * BLK
            pltpu.async_copy(src_hbm.at[pl.ds(off, BLK)], srcb[b], sem_i[b])
            pltpu.async_copy(dst_hbm.at[pl.ds(off, BLK)], dstb[b], sem_i[b])

        def wait_idx(j, b):
            off = e0 + j * BLK
            pltpu.make_async_copy(
                src_hbm.at[pl.ds(off, BLK)], srcb[b], sem_i[b]).wait()
            pltpu.make_async_copy(
                dst_hbm.at[pl.ds(off, BLK)], dstb[b], sem_i[b]).wait()

        def copy_dst(b):
            for k in range(BLK // 16):
                sl = pl.ds(16 * k, 16)
                dstp[b][sl] = dstb[b][sl]

        def issue_gather(b):
            pltpu.async_copy(u_hbm.at[srcb[b]], rows[b], sem_g[b])

        def wait_gather(b):
            pltpu.make_async_copy(u_hbm.at[srcb[b]], rows[b], sem_g[b]).wait()

        def issue_scatter(b):
            pltpu.async_copy(rows[b], acc.at[dstp[b]], sem_s[b], add=True)

        def wait_scatter(b):
            pltpu.make_async_copy(rows[b], acc.at[dstp[b]], sem_s[b]).wait()

        def step(j, b, *, first=False, last=False):
            if not first:
                wait_scatter(b)
            wait_idx(j, b)
            copy_dst(b)
            issue_gather(b)
            wait_gather(1 - b)
            issue_scatter(1 - b)
            if not last:
                issue_idx(j + 1, 1 - b)

        issue_idx(0, 0)
        wait_idx(0, 0)
        copy_dst(0)
        issue_gather(0)
        issue_idx(1, 1)
        step(1, 1, first=True)

        @pl.loop(0, (NFULL - 4) // 2)
        def _(i):
            j = 2 * i + 2
            step(j, 0)
            step(j + 1, 1)

        step(NFULL - 2, 0)
        step(NFULL - 1, 1, last=True)
        wait_gather(1)
        issue_scatter(1)
        wait_scatter(0)
        wait_scatter(1)

        off = e0 + NFULL * BLK
        pltpu.sync_copy(src_hbm.at[pl.ds(off, REM)], src_r)
        pltpu.sync_copy(dst_hbm.at[pl.ds(off, REM)], dst_r)
        pltpu.async_copy(
            u_hbm.at[src_r], rows_a.at[pl.ds(0, REM)], sem_ga).wait()
        pltpu.sync_copy(rows_a.at[pl.ds(0, REM)], acc.at[dst_r], add=True)

    @pl.when(c == 0)
    def _():
        run(u0)

    @pl.when(c == 1)
    def _():
        run(u1)

    plsc.subcore_barrier()

    def drain(out_ref):
        def fn(base, nrows):
            for off, k in chunks(nrows):
                rsl = pl.ds(base + off, k)
                pltpu.sync_copy(acc.at[rsl], rows_v.at[pl.ds(0, k)])
                pltpu.sync_copy(rows_v.at[pl.ds(0, k)], out_ref.at[rsl])

        rows_foreach(fn)

    @pl.when(c == 0)
    def _():
        drain(out0)

    @pl.when(c == 1)
    def _():
        drain(out1)


@functools.lru_cache(maxsize=None)
def _agg_call_built():
    return pl.kernel(
        _agg_body,
        out_type=[jax.ShapeDtypeStruct((N, HALF), _f32),
                  jax.ShapeDtypeStruct((N, HALF), _f32)],
        mesh=_sc_mesh(),
        scratch_types=[
            pltpu.VMEM_SHARED((N, HALF), _f32),
            pltpu.VMEM((BLK,), jnp.int32),
            pltpu.VMEM((BLK,), jnp.int32),
            pltpu.VMEM((BLK,), jnp.int32),
            pltpu.VMEM((BLK,), jnp.int32),
            pltpu.VMEM((BLK,), jnp.int32),
            pltpu.VMEM((BLK,), jnp.int32),
            pltpu.VMEM((BLK, HALF), _f32),
            pltpu.VMEM((BLK, HALF), _f32),
            pltpu.VMEM((REM,), jnp.int32),
            pltpu.VMEM((REM,), jnp.int32),
            pltpu.SemaphoreType.DMA,
            pltpu.SemaphoreType.DMA,
            pltpu.SemaphoreType.DMA,
            pltpu.SemaphoreType.DMA,
            pltpu.SemaphoreType.DMA,
            pltpu.SemaphoreType.DMA,
        ],
    )


def _agg_call(*args):
    return _agg_call_built()(*args)


_DOT = functools.partial(
    lax.dot_general,
    dimension_numbers=(((1,), (0,)), ((), ())),
    precision=lax.Precision.DEFAULT,
    preferred_element_type=_f32,
)


def _mm1_body(x_ref, w_ref, o_ref):
    o_ref[...] = _DOT(x_ref[...], w_ref[...])


def _mm1(x, w):
    return pl.pallas_call(
        _mm1_body,
        grid=(GRID,),
        in_specs=[pl.BlockSpec((ROW_BLK, IN_CH), lambda i: (i, 0)),
                  pl.BlockSpec((IN_CH, HID), lambda i: (0, 0))],
        out_specs=pl.BlockSpec((ROW_BLK, HID), lambda i: (i, 0)),
        out_shape=jax.ShapeDtypeStruct((N, HID), _f32),
    )(x, w)


def _scale_body(d0_ref, d1_ref, v_ref, dis_ref, u0_ref, u1_ref):
    deg = d0_ref[...] + d1_ref[...]
    dis = jnp.where(deg > 0, lax.rsqrt(jnp.maximum(deg, 1e-12)), 0.0)
    dis_ref[...] = dis
    u = dis * v_ref[...]
    u0_ref[...] = u[:, :HALF]
    u1_ref[...] = u[:, HALF:]


def _scale(deg0, deg1, v):
    return pl.pallas_call(
        _scale_body,
        grid=(GRID,),
        in_specs=[pl.BlockSpec((ROW_BLK, 1), lambda i: (i, 0)),
                  pl.BlockSpec((ROW_BLK, 1), lambda i: (i, 0)),
                  pl.BlockSpec((ROW_BLK, HID), lambda i: (i, 0))],
        out_specs=[pl.BlockSpec((ROW_BLK, 1), lambda i: (i, 0)),
                   pl.BlockSpec((ROW_BLK, HALF), lambda i: (i, 0)),
                   pl.BlockSpec((ROW_BLK, HALF), lambda i: (i, 0))],
        out_shape=[jax.ShapeDtypeStruct((N, 1), _f32),
                   jax.ShapeDtypeStruct((N, HALF), _f32),
                   jax.ShapeDtypeStruct((N, HALF), _f32)],
    )(deg0, deg1, v)


def _elu(x):
    return jnp.where(x > 0, x, jnp.exp(jnp.minimum(x, 0.0)) - 1.0)


def _mid_body(a0_ref, a1_ref, dis_ref, b_ref, w_ref, u0_ref, u1_ref):
    dis = dis_ref[...]
    h = jnp.concatenate([a0_ref[...], a1_ref[...]], axis=1)
    h = _elu(dis * h + b_ref[...])
    u = dis * _DOT(h, w_ref[...])
    u0_ref[...] = u[:, :HALF]
    u1_ref[...] = u[:, HALF:]


def _mid(a0, a1, dis, b, w):
    return pl.pallas_call(
        _mid_body,
        grid=(GRID,),
        in_specs=[pl.BlockSpec((ROW_BLK, HALF), lambda i: (i, 0)),
                  pl.BlockSpec((ROW_BLK, HALF), lambda i: (i, 0)),
                  pl.BlockSpec((ROW_BLK, 1), lambda i: (i, 0)),
                  pl.BlockSpec((1, HID), lambda i: (0, 0)),
                  pl.BlockSpec((HID, HID), lambda i: (0, 0))],
        out_specs=[pl.BlockSpec((ROW_BLK, HALF), lambda i: (i, 0)),
                   pl.BlockSpec((ROW_BLK, HALF), lambda i: (i, 0))],
        out_shape=[jax.ShapeDtypeStruct((N, HALF), _f32),
                   jax.ShapeDtypeStruct((N, HALF), _f32)],
    )(a0, a1, dis, b, w)


def _final_body(a0_ref, a1_ref, dis_ref, b_ref, o_ref):
    dis = dis_ref[...]
    h = jnp.concatenate([a0_ref[...], a1_ref[...]], axis=1)
    o_ref[...] = _elu(dis * h + b_ref[...])


def _final(a0, a1, dis, b):
    return pl.pallas_call(
        _final_body,
        grid=(GRID,),
        in_specs=[pl.BlockSpec((ROW_BLK, HALF), lambda i: (i, 0)),
                  pl.BlockSpec((ROW_BLK, HALF), lambda i: (i, 0)),
                  pl.BlockSpec((ROW_BLK, 1), lambda i: (i, 0)),
                  pl.BlockSpec((1, HID), lambda i: (0, 0))],
        out_specs=pl.BlockSpec((ROW_BLK, HID), lambda i: (i, 0)),
        out_shape=jax.ShapeDtypeStruct((N, HID), _f32),
    )(a0, a1, dis, b)


def kernel(x, edge_index, W1, b1, W2, b2, W3, b3):
    ei = edge_index.astype(jnp.int32)
    src = ei[0]
    dst = ei[1]
    zeros1 = jnp.zeros((N,), _f32)
    zeros2 = jnp.zeros((N, HALF), _f32)

    deg0, deg1 = _deg_call(dst, zeros1)
    v1 = _mm1(x, W1)
    dis, u0, u1 = _scale(deg0.reshape(N, 1), deg1.reshape(N, 1), v1)

    a0, a1 = _agg_call(u0, u1, src, dst, zeros2)
    u0, u1 = _mid(a0, a1, dis, b1.reshape(1, HID), W2)
    a0, a1 = _agg_call(u0, u1, src, dst, zeros2)
    u0, u1 = _mid(a0, a1, dis, b2.reshape(1, HID), W3)
    a0, a1 = _agg_call(u0, u1, src, dst, zeros2)
    return _final(a0, a1, dis, b3.reshape(1, HID))
